# Initial kernel scaffold; baseline (speedup 1.0000x reference)
import jax
import jax.numpy as jnp
from jax.experimental import pallas as pl


def kernel(idx_input, table):
    raise NotImplementedError("write your pallas kernel here")


# speedup vs baseline: 5.8074x; 5.8074x over previous
import functools

import jax
import jax.numpy as jnp
from jax import lax
from jax.experimental import pallas as pl
from jax.experimental.pallas import tpu as pltpu
from jax.experimental.pallas import tpu_sc as plsc

VOCAB_ROWS = 100002
DIM = 64
B = 16384
L = 50
N = B * L

NUM_CORES = 2
NUM_SUBCORES = 16
NW = NUM_CORES * NUM_SUBCORES
PER_W = N // NW
CHUNK = 512
NCHUNK = PER_W // CHUNK


def _make_kernel():
  mesh = plsc.VectorSubcoreMesh(core_axis_name="c", subcore_axis_name="s")

  @functools.partial(
      pl.kernel,
      mesh=mesh,
      compiler_params=pltpu.CompilerParams(use_tc_tiling_on_sc=False),
      out_type=jax.ShapeDtypeStruct((N, DIM), jnp.float32),
      scratch_types=[
          pltpu.VMEM((CHUNK,), jnp.int32),
          pltpu.VMEM((CHUNK, DIM), jnp.float32),
          pltpu.SemaphoreType.DMA,
      ],
  )
  def gather_kernel(idx_hbm, table_hbm, out_hbm, idx_v, rows_v, sem):
    wid = lax.axis_index("s") * NUM_CORES + lax.axis_index("c")
    base = wid * PER_W

    def body(g, carry):
      off = base + g * CHUNK
      pltpu.sync_copy(idx_hbm.at[pl.ds(off, CHUNK)], idx_v)
      pltpu.async_copy(table_hbm.at[idx_v], rows_v, sem).wait()
      pltpu.sync_copy(rows_v, out_hbm.at[pl.ds(off, CHUNK)])
      return carry

    lax.fori_loop(0, NCHUNK, body, 0)

  return gather_kernel


_gather = _make_kernel()


@jax.jit
def kernel(idx_input, table):
  idx_flat = idx_input.reshape(-1).astype(jnp.int32)
  out = _gather(idx_flat, table)
  return out.reshape(B, L, DIM)


# speedup vs baseline: 6.1674x; 1.0620x over previous
import functools

import jax
import jax.numpy as jnp
from jax import lax
from jax.experimental import pallas as pl
from jax.experimental.pallas import tpu as pltpu
from jax.experimental.pallas import tpu_sc as plsc

VOCAB_ROWS = 100002
DIM = 64
B = 16384
L = 50
N = B * L

NUM_CORES = 2
NUM_SUBCORES = 16
NW = NUM_CORES * NUM_SUBCORES
PER_W = N // NW
CHUNK = 640
NCHUNK = PER_W // CHUNK
NPAIR = NCHUNK // 2


def _make_kernel():
  mesh = plsc.VectorSubcoreMesh(core_axis_name="c", subcore_axis_name="s")

  @functools.partial(
      pl.kernel,
      mesh=mesh,
      compiler_params=pltpu.CompilerParams(use_tc_tiling_on_sc=False),
      out_type=jax.ShapeDtypeStruct((N, DIM), jnp.float32),
      scratch_types=[
          pltpu.VMEM((PER_W,), jnp.int32),
          pltpu.VMEM((CHUNK, DIM), jnp.float32),
          pltpu.VMEM((CHUNK, DIM), jnp.float32),
          pltpu.SemaphoreType.DMA,
          pltpu.SemaphoreType.DMA,
          pltpu.SemaphoreType.DMA,
          pltpu.SemaphoreType.DMA,
      ],
  )
  def gather_kernel(idx_hbm, table_hbm, out_hbm, idx_v, rows0, rows1,
                    gsem0, gsem1, wsem0, wsem1):
    wid = lax.axis_index("s") * NUM_CORES + lax.axis_index("c")
    base = wid * PER_W
    pltpu.sync_copy(idx_hbm.at[pl.ds(base, PER_W)], idx_v)

    def start_gather(g, buf, sem):
      pltpu.async_copy(table_hbm.at[idx_v.at[pl.ds(g * CHUNK, CHUNK)]],
                       buf, sem)

    def wait_gather(g, buf, sem):
      pltpu.make_async_copy(
          table_hbm.at[idx_v.at[pl.ds(g * CHUNK, CHUNK)]], buf, sem).wait()

    def start_write(g, buf, sem):
      pltpu.async_copy(buf, out_hbm.at[pl.ds(base + g * CHUNK, CHUNK)], sem)

    def wait_write(g, buf, sem):
      pltpu.make_async_copy(
          buf, out_hbm.at[pl.ds(base + g * CHUNK, CHUNK)], sem).wait()

    start_gather(0, rows0, gsem0)
    start_gather(1, rows1, gsem1)

    def body(i, carry):
      g0 = 2 * i
      g1 = g0 + 1
      wait_gather(g0, rows0, gsem0)
      start_write(g0, rows0, wsem0)
      wait_gather(g1, rows1, gsem1)
      start_write(g1, rows1, wsem1)
      wait_write(g0, rows0, wsem0)
      start_gather(g0 + 2, rows0, gsem0)
      wait_write(g1, rows1, wsem1)
      start_gather(g1 + 2, rows1, gsem1)
      return carry

    lax.fori_loop(0, NPAIR - 1, body, 0)

    gl0 = NCHUNK - 2
    gl1 = NCHUNK - 1
    wait_gather(gl0, rows0, gsem0)
    start_write(gl0, rows0, wsem0)
    wait_gather(gl1, rows1, gsem1)
    start_write(gl1, rows1, wsem1)
    wait_write(gl0, rows0, wsem0)
    wait_write(gl1, rows1, wsem1)

  return gather_kernel


_gather = _make_kernel()


@jax.jit
def kernel(idx_input, table):
  idx_flat = idx_input.reshape(-1).astype(jnp.int32)
  out = _gather(idx_flat, table)
  return out.reshape(B, L, DIM)


# speedup vs baseline: 6.1688x; 1.0002x over previous
import functools

import jax
import jax.numpy as jnp
from jax import lax
from jax._src.layout import Format, Layout
from jax.experimental import pallas as pl
from jax.experimental.pallas import tpu as pltpu
from jax.experimental.pallas import tpu_sc as plsc

VOCAB_ROWS = 100002
DIM = 64
B = 16384
L = 50
N = B * L

NUM_CORES = 2
NUM_SUBCORES = 16
NW = NUM_CORES * NUM_SUBCORES
PER_W = N // NW
CHUNK = 640
NCHUNK = PER_W // CHUNK
NPAIR = NCHUNK // 2


def _make_kernel():
  mesh = plsc.VectorSubcoreMesh(core_axis_name="c", subcore_axis_name="s")

  @functools.partial(
      pl.kernel,
      mesh=mesh,
      compiler_params=pltpu.CompilerParams(use_tc_tiling_on_sc=False),
      out_type=jax.ShapeDtypeStruct((N, DIM), jnp.float32),
      scratch_types=[
          pltpu.VMEM((PER_W,), jnp.int32),
          pltpu.VMEM((CHUNK, DIM), jnp.float32),
          pltpu.VMEM((CHUNK, DIM), jnp.float32),
          pltpu.SemaphoreType.DMA,
          pltpu.SemaphoreType.DMA,
          pltpu.SemaphoreType.DMA,
          pltpu.SemaphoreType.DMA,
      ],
  )
  def gather_kernel(idx_hbm, table_hbm, out_hbm, idx_v, rows0, rows1,
                    gsem0, gsem1, wsem0, wsem1):
    wid = lax.axis_index("s") * NUM_CORES + lax.axis_index("c")
    base = wid * PER_W
    pltpu.sync_copy(idx_hbm.at[pl.ds(base, PER_W)], idx_v)

    def start_gather(g, buf, sem):
      pltpu.async_copy(table_hbm.at[idx_v.at[pl.ds(g * CHUNK, CHUNK)]],
                       buf, sem)

    def wait_gather(g, buf, sem):
      pltpu.make_async_copy(
          table_hbm.at[idx_v.at[pl.ds(g * CHUNK, CHUNK)]], buf, sem).wait()

    def start_write(g, buf, sem):
      pltpu.async_copy(buf, out_hbm.at[pl.ds(base + g * CHUNK, CHUNK)], sem)

    def wait_write(g, buf, sem):
      pltpu.make_async_copy(
          buf, out_hbm.at[pl.ds(base + g * CHUNK, CHUNK)], sem).wait()

    start_gather(0, rows0, gsem0)
    start_gather(1, rows1, gsem1)

    def body(i, carry):
      g0 = 2 * i
      g1 = g0 + 1
      wait_gather(g0, rows0, gsem0)
      start_write(g0, rows0, wsem0)
      wait_gather(g1, rows1, gsem1)
      start_write(g1, rows1, wsem1)
      wait_write(g0, rows0, wsem0)
      start_gather(g0 + 2, rows0, gsem0)
      wait_write(g1, rows1, wsem1)
      start_gather(g1 + 2, rows1, gsem1)
      return carry

    lax.fori_loop(0, NPAIR - 1, body, 0)

    gl0 = NCHUNK - 2
    gl1 = NCHUNK - 1
    wait_gather(gl0, rows0, gsem0)
    start_write(gl0, rows0, wsem0)
    wait_gather(gl1, rows1, gsem1)
    start_write(gl1, rows1, wsem1)
    wait_write(gl0, rows0, wsem0)
    wait_write(gl1, rows1, wsem1)

  return gather_kernel


_gather = _make_kernel()


def _impl(idx_input, table):
  idx_flat = idx_input.reshape(-1).astype(jnp.int32)
  out = _gather(idx_flat, table)
  return out.reshape(B, L, DIM)


_jitted = None


def kernel(idx_input, table):
  global _jitted
  if _jitted is None:
    try:
      dev = next(iter(idx_input.devices()))
    except Exception:
      dev = jax.devices()[0]
    fmt = Format(Layout(major_to_minor=(0, 1, 2)),
                 jax.sharding.SingleDeviceSharding(dev))
    _jitted = jax.jit(_impl, out_shardings=fmt)
  return _jitted(idx_input, table)
